# baseline (speedup 1.0000x reference)
import functools
import math

import jax
import jax.numpy as jnp
from jax import lax
from jax.experimental import pallas as pl
from jax.experimental.pallas import tpu as pltpu
from jax.experimental.pallas import tpu_sc as plsc

_L = 16
_NW = 32
_BR = 80
_CHUNK = 128


def _mesh():
    return plsc.VectorSubcoreMesh(core_axis_name="c", subcore_axis_name="s")


@functools.lru_cache(maxsize=None)
def _build_compact(vocab: int, dim: int):
    assert vocab % (2 * _BR) == 0
    nblocks = vocab // _BR
    pairs_per_block = _BR // 2

    @functools.partial(
        pl.kernel,
        out_type=jax.ShapeDtypeStruct((vocab // 2, 2 * dim), jnp.float32),
        mesh=_mesh(),
        scratch_types=[
            pltpu.VMEM((2, _BR, dim), jnp.float32),
            pltpu.VMEM((2, pairs_per_block, 2 * dim), jnp.float32),
            pltpu.SemaphoreType.DMA,
            pltpu.SemaphoreType.DMA,
        ],
    )
    def compact(table_hbm, out_hbm, abuf, bbuf, rsem, wsem):
        wid = lax.axis_index("s") * 2 + lax.axis_index("c")
        nt = (nblocks - wid + _NW - 1) // _NW

        def read(b, slot):
            return pltpu.make_async_copy(
                table_hbm.at[pl.ds(pl.multiple_of(b * _BR, 8), _BR)],
                abuf.at[slot], rsem)

        def write(b, slot):
            return pltpu.make_async_copy(
                bbuf.at[slot],
                out_hbm.at[pl.ds(pl.multiple_of(b * pairs_per_block, 8),
                                 pairs_per_block)], wsem)

        read(wid, 0).start()
        read(wid + _NW, 1).start()

        def step(t, carry):
            b = wid + t * _NW
            slot = lax.rem(t, 2)
            read(b, slot).wait()

            @pl.when(t >= 2)
            def _():
                write(b, slot).wait()

            def pair(p, c2):
                for k in range(dim // _L):
                    sl = pl.ds(k * _L, _L)
                    bbuf[slot, p, sl] = abuf[slot, 2 * p, sl]
                    bbuf[slot, p, pl.ds(dim + k * _L, _L)] = \
                        abuf[slot, 2 * p + 1, sl]
                return c2
            lax.fori_loop(0, pairs_per_block, pair, 0)

            write(b, slot).start()

            @pl.when(t + 2 < nt)
            def _():
                read(b + 2 * _NW, slot).start()
            return carry

        lax.fori_loop(0, nt, step, 0)
        write(0, lax.rem(nt - 2, 2)).wait()
        write(0, lax.rem(nt - 1, 2)).wait()

    return compact


@functools.lru_cache(maxsize=None)
def _build_lookup(seq: int, batch: int, vocab: int, dim: int):
    assert batch == _CHUNK * _NW
    scale = math.sqrt(dim)
    nk = dim // _L

    n_chunks = (seq * batch) // _CHUNK
    cpw = n_chunks // _NW
    cps = batch // _CHUNK

    @functools.partial(
        pl.kernel,
        out_type=jax.ShapeDtypeStruct((seq, batch, dim), jnp.float32),
        mesh=_mesh(),
        scratch_types=[
            pltpu.VMEM((cpw, _CHUNK), jnp.int32),
            pltpu.VMEM((cpw, _CHUNK), jnp.int32),
            pltpu.VMEM((16, dim), jnp.float32),
            pltpu.VMEM((2, _CHUNK, 2 * dim), jnp.float32),
            pltpu.VMEM((2, _CHUNK, dim), jnp.float32),
            pltpu.SemaphoreType.DMA,
            pltpu.SemaphoreType.DMA,
            pltpu.SemaphoreType.DMA,
        ],
    )
    def lookup(x_hbm, tc_hbm, pe_hbm, out_hbm,
               idx_v, pidx_v, pe_v, gbuf, obuf, ssem, gsem, osem):
        wid = lax.axis_index("s") * 2 + lax.axis_index("c")
        base_c = pl.multiple_of(wid * cpw, 8)
        s0 = base_c // cps
        start8 = pl.multiple_of(
            lax.min((s0 // 8) * 8, jnp.int32(seq - 16)), 8)

        pltpu.make_async_copy(x_hbm.at[pl.ds(base_c, cpw)], idx_v,
                              ssem).start()
        pltpu.make_async_copy(pe_hbm.at[pl.ds(start8, 16)], pe_v,
                              ssem).start()
        pltpu.make_async_copy(x_hbm.at[pl.ds(base_c, cpw)], idx_v,
                              ssem).wait()
        pltpu.make_async_copy(pe_hbm.at[pl.ds(start8, 16)], pe_v,
                              ssem).wait()

        def shift(i, c2):
            for k in range(_CHUNK // _L):
                sl = pl.ds(k * _L, _L)
                pidx_v[i, sl] = lax.shift_right_logical(idx_v[i, sl], 1)
            return c2
        lax.fori_loop(0, cpw, shift, 0)

        def gather(t, slot):
            return pltpu.make_async_copy(
                tc_hbm.at[pidx_v.at[t]], gbuf.at[slot], gsem)

        def put(t, slot):
            c = base_c + t
            return pltpu.make_async_copy(
                obuf.at[slot],
                out_hbm.at[c // cps,
                           pl.ds(pl.multiple_of(lax.rem(c, cps) * _CHUNK, 8),
                                 _CHUNK)],
                osem)

        gather(0, 0).start()
        gather(1, 1).start()

        def step(t, carry):
            slot = lax.rem(t, 2)
            gather(t, slot).wait()

            @pl.when(t >= 2)
            def _():
                put(t, slot).wait()

            s_loc = (base_c + t) // cps - start8
            pe_regs = [pe_v[s_loc, pl.ds(k * _L, _L)] for k in range(nk)]

            def rowgroup(g, c2):
                rbase = g * _L
                halves = lax.bitwise_and(idx_v[t, pl.ds(rbase, _L)], 1) * dim
                for i in range(_L):
                    half = halves[i]
                    for k in range(nk):
                        v = gbuf[slot, rbase + i, pl.ds(half + k * _L, _L)]
                        obuf[slot, rbase + i, pl.ds(k * _L, _L)] = \
                            v * scale + pe_regs[k]
                return c2
            lax.fori_loop(0, _CHUNK // _L, rowgroup, 0)

            put(t, slot).start()

            @pl.when(t + 2 < cpw)
            def _():
                gather(t + 2, slot).start()
            return carry

        lax.fori_loop(0, cpw, step, 0)
        put(cpw - 2, lax.rem(cpw - 2, 2)).wait()
        put(cpw - 1, lax.rem(cpw - 1, 2)).wait()

    return lookup


def kernel(x, table, pe, pos):
    seq, batch = x.shape
    vocab, dim = table.shape
    tc = _build_compact(vocab, dim)(table)
    pe_rows = lax.dynamic_slice_in_dim(pe, pos, seq, axis=0).reshape(seq, dim)
    x2 = x.astype(jnp.int32).reshape((seq * batch) // _CHUNK, _CHUNK)
    return _build_lookup(seq, batch, vocab, dim)(x2, tc, pe_rows)
